# baseline (speedup 1.0000x reference)
import jax
import jax.numpy as jnp
from jax import lax
from jax.experimental import pallas as pl
from jax.experimental.pallas import tpu as pltpu

BATCH = 8
SEQ = 12
FEAT = 64
NODES = 128
KS = 3
DILS = (1, 2)
T0 = 5
NT = SEQ - T0
BT = BATCH * SEQ

_HI = lax.Precision.HIGHEST


def _shift_mat(s):
    r = lax.broadcasted_iota(jnp.int32, (BT, BT), 0)
    c = lax.broadcasted_iota(jnp.int32, (BT, BT), 1)
    return ((r - c == BATCH * s) & (r >= BATCH * s)).astype(jnp.float32)


def _tap(cw, k):
    lane = lax.broadcasted_iota(jnp.int32, (FEAT, FEAT, KS), 2)
    return jnp.sum(jnp.where(lane == k, cw, 0.0), axis=-1)


def _fused_kernel(ne_ref, w0_ref, b0_ref, w1_ref, b1_ref,
                  cw0_ref, cb0_ref, cw1_ref, cb1_ref, out_ref, mu_ref):
    i = pl.program_id(0)

    @pl.when(i == 0)
    def _zero():
        mu_ref[...] = jnp.zeros((BT, FEAT), jnp.float32)

    mu_t = jnp.mean(ne_ref[:, 0], axis=-1)
    mu_ref[pl.ds((i + T0) * BATCH, BATCH), :] = mu_t

    @pl.when(i == NT - 1)
    def _finish():
        mu0 = mu_ref[...]
        g0 = jnp.dot(mu0, w0_ref[...], precision=_HI) + b0_ref[...]

        def causal_conv(g, cw_ref, cb_ref, d):
            cw = cw_ref[...]
            acc = jnp.zeros((BT, FEAT), jnp.float32)
            for k in range(KS):
                s = (KS - 1 - k) * d
                gs = g if s == 0 else jnp.dot(_shift_mat(s), g, precision=_HI)
                acc = acc + lax.dot_general(
                    gs, _tap(cw, k), (((1,), (1,)), ((), ())), precision=_HI)
            return jax.nn.relu(acc + cb_ref[...])

        c0 = causal_conv(g0, cw0_ref, cb0_ref, DILS[0])
        mu1 = mu0 + c0
        g1 = jnp.dot(mu1, w1_ref[...], precision=_HI) + b1_ref[...]
        c1 = causal_conv(g1, cw1_ref, cb1_ref, DILS[1])

        rp = lax.broadcasted_iota(jnp.int32, (BATCH, BT), 0)
        cp = lax.broadcasted_iota(jnp.int32, (BATCH, BT), 1)
        P = (cp == (SEQ - 1) * BATCH + rp).astype(jnp.float32)
        y = 0.5 * jnp.dot(P, c0 + c1, precision=_HI)
        out_ref[...] = jnp.broadcast_to(y[:, None, :], (BATCH, NODES, FEAT))


def kernel(node_embeddings, B, static_MTE_matrix, batch_index, use_MTE,
           is_training, learnable_adj, W_gcn0, b_gcn0, W_gcn1, b_gcn1,
           conv_w0, conv_b0, conv_w1, conv_b1, Wa, ba, v):
    cw0 = conv_w0.reshape(FEAT, FEAT, KS)
    cw1 = conv_w1.reshape(FEAT, FEAT, KS)
    b0 = b_gcn0.reshape(1, FEAT)
    b1 = b_gcn1.reshape(1, FEAT)
    cb0 = conv_b0.reshape(1, FEAT)
    cb1 = conv_b1.reshape(1, FEAT)

    full = lambda shape: pl.BlockSpec(shape, lambda i: (0,) * len(shape))
    out = pl.pallas_call(
        _fused_kernel,
        grid=(NT,),
        in_specs=[
            pl.BlockSpec((BATCH, 1, FEAT, NODES), lambda i: (0, i + T0, 0, 0)),
            full((FEAT, FEAT)), full((1, FEAT)),
            full((FEAT, FEAT)), full((1, FEAT)),
            full((FEAT, FEAT, KS)), full((1, FEAT)),
            full((FEAT, FEAT, KS)), full((1, FEAT)),
        ],
        out_specs=pl.BlockSpec((BATCH, NODES, FEAT), lambda i: (0, 0, 0)),
        out_shape=jax.ShapeDtypeStruct((BATCH, NODES, FEAT), jnp.float32),
        scratch_shapes=[pltpu.VMEM((BT, FEAT), jnp.float32)],
    )(node_embeddings, W_gcn0, b0, W_gcn1, b1, cw0, cb0, cw1, cb1)
    return out


# speedup vs baseline: 1.0010x; 1.0010x over previous
import jax
import jax.numpy as jnp
from jax import lax
from jax.experimental import pallas as pl

BATCH = 8
SEQ = 12
FEAT = 64
NODES = 128
KS = 3
DILS = (1, 2)
BT = BATCH * SEQ

_HI = lax.Precision.HIGHEST


def _shift_mat(s):
    r = lax.broadcasted_iota(jnp.int32, (BT, BT), 0)
    c = lax.broadcasted_iota(jnp.int32, (BT, BT), 1)
    return ((r - c == s) & (r % SEQ >= s)).astype(jnp.float32)


def _tap(cw, k):
    lane = lax.broadcasted_iota(jnp.int32, (FEAT, FEAT, KS), 2)
    return jnp.sum(jnp.where(lane == k, cw, 0.0), axis=-1)


def _fused_kernel(ne_ref, w0_ref, b0_ref, w1_ref, b1_ref,
                  cw0_ref, cb0_ref, cw1_ref, cb1_ref, out_ref):
    mu0 = jnp.mean(ne_ref[...], axis=-1)
    g0 = jnp.dot(mu0, w0_ref[...], precision=_HI) + b0_ref[...]

    def causal_conv(g, cw_ref, cb_ref, d):
        cw = cw_ref[...]
        acc = jnp.zeros((BT, FEAT), jnp.float32)
        for k in range(KS):
            s = (KS - 1 - k) * d
            gs = g if s == 0 else jnp.dot(_shift_mat(s), g, precision=_HI)
            acc = acc + lax.dot_general(
                gs, _tap(cw, k), (((1,), (1,)), ((), ())), precision=_HI)
        return jax.nn.relu(acc + cb_ref[...])

    c0 = causal_conv(g0, cw0_ref, cb0_ref, DILS[0])
    mu1 = mu0 + c0
    g1 = jnp.dot(mu1, w1_ref[...], precision=_HI) + b1_ref[...]
    c1 = causal_conv(g1, cw1_ref, cb1_ref, DILS[1])

    rp = lax.broadcasted_iota(jnp.int32, (BATCH, BT), 0)
    cp = lax.broadcasted_iota(jnp.int32, (BATCH, BT), 1)
    P = (cp == rp * SEQ + (SEQ - 1)).astype(jnp.float32)
    y = 0.5 * jnp.dot(P, c0 + c1, precision=_HI)
    out_ref[...] = jnp.broadcast_to(y[:, None, :], (BATCH, NODES, FEAT))


def kernel(node_embeddings, B, static_MTE_matrix, batch_index, use_MTE,
           is_training, learnable_adj, W_gcn0, b_gcn0, W_gcn1, b_gcn1,
           conv_w0, conv_b0, conv_w1, conv_b1, Wa, ba, v):
    ne3 = node_embeddings.reshape(BT, FEAT, NODES)
    cw0 = conv_w0.reshape(FEAT, FEAT, KS)
    cw1 = conv_w1.reshape(FEAT, FEAT, KS)
    b0 = b_gcn0.reshape(1, FEAT)
    b1 = b_gcn1.reshape(1, FEAT)
    cb0 = conv_b0.reshape(1, FEAT)
    cb1 = conv_b1.reshape(1, FEAT)

    out = pl.pallas_call(
        _fused_kernel,
        out_shape=jax.ShapeDtypeStruct((BATCH, NODES, FEAT), jnp.float32),
    )(ne3, W_gcn0, b0, W_gcn1, b1, cw0, cb0, cw1, cb1)
    return out


# speedup vs baseline: 1.4593x; 1.4579x over previous
import numpy as np
import jax
import jax.numpy as jnp
from jax import lax
from jax.experimental import pallas as pl
from jax.experimental.pallas import tpu as pltpu

BATCH = 8
SEQ = 12
FEAT = 64
NODES = 128
KS = 3
DILS = (1, 2)
BT = BATCH * SEQ

_HI = lax.Precision.HIGHEST

_TIDX = np.arange(BT, dtype=np.float32).reshape(BT, 1) % SEQ
_PSEL = np.zeros((BATCH, BT), dtype=np.float32)
for _b in range(BATCH):
    _PSEL[_b, _b * SEQ + (SEQ - 1)] = 0.5


def _fused_kernel(ne_ref, w0_ref, b0_ref, w1_ref, b1_ref,
                  cw0_ref, cb0_ref, cw1_ref, cb1_ref, tidx_ref, psel_ref,
                  out_ref):
    tidx = tidx_ref[...]
    mu0 = jnp.mean(ne_ref[...], axis=-1)
    g0 = jnp.dot(mu0, w0_ref[...], precision=_HI) + b0_ref[...]

    def causal_conv(g, cw_ref, cb_ref, d):
        acc = jnp.zeros((BT, FEAT), jnp.float32)
        for k in range(KS):
            s = (KS - 1 - k) * d
            if s == 0:
                gs = g
            else:
                gs = jnp.where(tidx >= s, pltpu.roll(g, s, 0), 0.0)
            acc = acc + jnp.dot(gs, cw_ref[k], precision=_HI)
        return jax.nn.relu(acc + cb_ref[...])

    c0 = causal_conv(g0, cw0_ref, cb0_ref, DILS[0])
    mu1 = mu0 + c0
    g1 = jnp.dot(mu1, w1_ref[...], precision=_HI) + b1_ref[...]
    c1 = causal_conv(g1, cw1_ref, cb1_ref, DILS[1])

    y = jnp.dot(psel_ref[...], c0 + c1, precision=_HI)
    out_ref[...] = jnp.broadcast_to(y[:, None, :], (BATCH, NODES, FEAT))


def kernel(node_embeddings, B, static_MTE_matrix, batch_index, use_MTE,
           is_training, learnable_adj, W_gcn0, b_gcn0, W_gcn1, b_gcn1,
           conv_w0, conv_b0, conv_w1, conv_b1, Wa, ba, v):
    ne3 = node_embeddings.reshape(BT, FEAT, NODES)
    cw0m = jnp.transpose(conv_w0[:, :, 0, :], (2, 1, 0))
    cw1m = jnp.transpose(conv_w1[:, :, 0, :], (2, 1, 0))
    b0 = b_gcn0.reshape(1, FEAT)
    b1 = b_gcn1.reshape(1, FEAT)
    cb0 = conv_b0.reshape(1, FEAT)
    cb1 = conv_b1.reshape(1, FEAT)

    out = pl.pallas_call(
        _fused_kernel,
        out_shape=jax.ShapeDtypeStruct((BATCH, NODES, FEAT), jnp.float32),
    )(ne3, W_gcn0, b0, W_gcn1, b1, cw0m, cb0, cw1m, cb1,
      jnp.asarray(_TIDX), jnp.asarray(_PSEL))
    return out


# speedup vs baseline: 1.8522x; 1.2692x over previous
import numpy as np
import jax
import jax.numpy as jnp
from jax import lax
from jax.experimental import pallas as pl
from jax.experimental.pallas import tpu as pltpu

BATCH = 8
SEQ = 12
FEAT = 64
NODES = 128
KS = 3
DILS = (1, 2)
T0 = 4
NT = SEQ - T0
RR = BATCH * NT

_HI = lax.Precision.HIGHEST

_TIDX = np.arange(RR, dtype=np.float32).reshape(RR, 1) % NT
_PSEL = np.zeros((BATCH, RR), dtype=np.float32)
for _b in range(BATCH):
    _PSEL[_b, _b * NT + (NT - 1)] = 0.5


def _fused_kernel(nea_ref, neb_ref, w0_ref, b0_ref, w1_ref, b1_ref,
                  cw0_ref, cb0_ref, cw1_ref, cb1_ref, tidx_ref, psel_ref,
                  out_ref):
    tidx = tidx_ref[...]
    mua = jnp.mean(nea_ref[...], axis=-1)
    mub = jnp.mean(neb_ref[...], axis=-1)
    mu0 = jnp.reshape(jnp.concatenate([mua, mub], axis=1), (RR, FEAT))
    g0 = jnp.dot(mu0, w0_ref[...], precision=_HI) + b0_ref[...]

    def causal_conv(g, cw_ref, cb_ref, d):
        acc = jnp.zeros((RR, FEAT), jnp.float32)
        for k in range(KS):
            s = (KS - 1 - k) * d
            if s == 0:
                gs = g
            else:
                gs = jnp.where(tidx >= s, pltpu.roll(g, s, 0), 0.0)
            acc = acc + jnp.dot(gs, cw_ref[k], precision=_HI)
        return jax.nn.relu(acc + cb_ref[...])

    c0 = causal_conv(g0, cw0_ref, cb0_ref, DILS[0])
    mu1 = mu0 + c0
    g1 = jnp.dot(mu1, w1_ref[...], precision=_HI) + b1_ref[...]
    c1 = causal_conv(g1, cw1_ref, cb1_ref, DILS[1])

    y = jnp.dot(psel_ref[...], c0 + c1, precision=_HI)
    out_ref[...] = jnp.broadcast_to(y[:, None, :], (BATCH, NODES, FEAT))


def kernel(node_embeddings, B, static_MTE_matrix, batch_index, use_MTE,
           is_training, learnable_adj, W_gcn0, b_gcn0, W_gcn1, b_gcn1,
           conv_w0, conv_b0, conv_w1, conv_b1, Wa, ba, v):
    cw0m = jnp.transpose(conv_w0[:, :, 0, :], (2, 1, 0))
    cw1m = jnp.transpose(conv_w1[:, :, 0, :], (2, 1, 0))
    b0 = b_gcn0.reshape(1, FEAT)
    b1 = b_gcn1.reshape(1, FEAT)
    cb0 = conv_b0.reshape(1, FEAT)
    cb1 = conv_b1.reshape(1, FEAT)

    half = NT // 2
    out = pl.pallas_call(
        _fused_kernel,
        grid=(1,),
        in_specs=[
            pl.BlockSpec((BATCH, half, FEAT, NODES), lambda i: (0, 1, 0, 0)),
            pl.BlockSpec((BATCH, half, FEAT, NODES), lambda i: (0, 2, 0, 0)),
            pl.BlockSpec((FEAT, FEAT), lambda i: (0, 0)),
            pl.BlockSpec((1, FEAT), lambda i: (0, 0)),
            pl.BlockSpec((FEAT, FEAT), lambda i: (0, 0)),
            pl.BlockSpec((1, FEAT), lambda i: (0, 0)),
            pl.BlockSpec((KS, FEAT, FEAT), lambda i: (0, 0, 0)),
            pl.BlockSpec((1, FEAT), lambda i: (0, 0)),
            pl.BlockSpec((KS, FEAT, FEAT), lambda i: (0, 0, 0)),
            pl.BlockSpec((1, FEAT), lambda i: (0, 0)),
            pl.BlockSpec((RR, 1), lambda i: (0, 0)),
            pl.BlockSpec((BATCH, RR), lambda i: (0, 0)),
        ],
        out_specs=pl.BlockSpec((BATCH, NODES, FEAT), lambda i: (0, 0, 0)),
        out_shape=jax.ShapeDtypeStruct((BATCH, NODES, FEAT), jnp.float32),
    )(node_embeddings, node_embeddings, W_gcn0, b0, W_gcn1, b1,
      cw0m, cb0, cw1m, cb1, jnp.asarray(_TIDX), jnp.asarray(_PSEL))
    return out
